# Initial kernel scaffold; baseline (speedup 1.0000x reference)
import jax
import jax.numpy as jnp
from jax.experimental import pallas as pl


def kernel(positions, pe):
    raise NotImplementedError("write your pallas kernel here")


# speedup vs baseline: 3.2086x; 3.2086x over previous
import functools

import jax
import jax.numpy as jnp
from jax import lax
from jax.experimental import pallas as pl
from jax.experimental.pallas import tpu as pltpu
from jax.experimental.pallas import tpu_sc as plsc

DIM = 128
N_EDGES = 320000
NUM_CORES = 2
NUM_SUBCORES = 16
LANES = 16
NUM_WORKERS = NUM_CORES * NUM_SUBCORES
ROWS_PER_WORKER = N_EDGES // NUM_WORKERS
CHUNK = 80
NUM_CHUNKS = ROWS_PER_WORKER // CHUNK


def _sc_gather(positions, pe):
    mesh = plsc.VectorSubcoreMesh(core_axis_name="c", subcore_axis_name="s")

    @functools.partial(
        pl.kernel,
        out_type=jax.ShapeDtypeStruct((N_EDGES, DIM), jnp.float32),
        mesh=mesh,
        scratch_types=[
            pltpu.VMEM((ROWS_PER_WORKER,), jnp.float32),
            pltpu.VMEM((ROWS_PER_WORKER,), jnp.int32),
            pltpu.VMEM((CHUNK, DIM), jnp.float32),
            pltpu.SemaphoreType.DMA,
        ],
    )
    def run(pos_hbm, pe_hbm, out_hbm, pos_v, idx_v, rows_v, sem):
        wid = lax.axis_index("s") * NUM_CORES + lax.axis_index("c")
        base = wid * ROWS_PER_WORKER
        pltpu.sync_copy(pos_hbm.at[pl.ds(base, ROWS_PER_WORKER)], pos_v)

        @pl.loop(0, ROWS_PER_WORKER, step=LANES)
        def _(i):
            p = pos_v[pl.ds(i, LANES)]
            v = jnp.minimum(jnp.maximum((p * 999.0).astype(jnp.int32), 0), 999)
            idx_v[pl.ds(i, LANES)] = v

        @pl.loop(0, NUM_CHUNKS)
        def _(c):
            off = c * CHUNK
            pltpu.async_copy(
                pe_hbm.at[idx_v.at[pl.ds(off, CHUNK)]], rows_v, sem
            ).wait()
            pltpu.sync_copy(rows_v, out_hbm.at[pl.ds(base + off, CHUNK)])

    return run(positions, pe)


def kernel(positions, pe):
    return _sc_gather(positions, pe)


# speedup vs baseline: 3.7927x; 1.1820x over previous
import functools

import jax
import jax.numpy as jnp
from jax import lax
from jax.experimental import pallas as pl
from jax.experimental.pallas import tpu as pltpu
from jax.experimental.pallas import tpu_sc as plsc

DIM = 128
N_EDGES = 320000
NUM_CORES = 2
NUM_SUBCORES = 16
LANES = 16
NUM_WORKERS = NUM_CORES * NUM_SUBCORES
ROWS_PER_WORKER = N_EDGES // NUM_WORKERS
CHUNK = 80
NUM_CHUNKS = ROWS_PER_WORKER // CHUNK
NBUF = 5
GROUPS = NUM_CHUNKS // NBUF


def _sc_gather(positions, pe):
    mesh = plsc.VectorSubcoreMesh(core_axis_name="c", subcore_axis_name="s")

    @functools.partial(
        pl.kernel,
        out_type=jax.ShapeDtypeStruct((N_EDGES, DIM), jnp.float32),
        mesh=mesh,
        scratch_types=[
            pltpu.VMEM((ROWS_PER_WORKER,), jnp.float32),
            pltpu.VMEM((ROWS_PER_WORKER,), jnp.int32),
            pltpu.VMEM((NBUF, CHUNK, DIM), jnp.float32),
            pltpu.SemaphoreType.DMA((NBUF,)),
            pltpu.SemaphoreType.DMA((NBUF,)),
        ],
    )
    def run(pos_hbm, pe_hbm, out_hbm, pos_v, idx_v, rows, gsem, wsem):
        wid = lax.axis_index("s") * NUM_CORES + lax.axis_index("c")
        base = wid * ROWS_PER_WORKER
        pltpu.sync_copy(pos_hbm.at[pl.ds(base, ROWS_PER_WORKER)], pos_v)

        @pl.loop(0, ROWS_PER_WORKER, step=LANES)
        def _(i):
            p = pos_v[pl.ds(i, LANES)]
            v = jnp.minimum(jnp.maximum((p * 999.0).astype(jnp.int32), 0), 999)
            idx_v[pl.ds(i, LANES)] = v

        def fire_gathers(g):
            return [
                pltpu.async_copy(
                    pe_hbm.at[idx_v.at[pl.ds((g * NBUF + b) * CHUNK, CHUNK)]],
                    rows.at[b],
                    gsem.at[b],
                )
                for b in range(NBUF)
            ]

        def drain_and_write(g, gathers):
            for b in range(NBUF):
                gathers[b].wait()
                pltpu.async_copy(
                    rows.at[b],
                    out_hbm.at[pl.ds(base + (g * NBUF + b) * CHUNK, CHUNK)],
                    wsem.at[b],
                )

        def drain_writes():
            for b in range(NBUF):
                pltpu.make_async_copy(
                    rows.at[b], out_hbm.at[pl.ds(base, CHUNK)], wsem.at[b]
                ).wait()

        drain_and_write(0, fire_gathers(0))

        @pl.loop(1, GROUPS)
        def _(g):
            drain_writes()
            drain_and_write(g, fire_gathers(g))

        drain_writes()

    return run(positions, pe)


def kernel(positions, pe):
    return _sc_gather(positions, pe)


# speedup vs baseline: 8.5415x; 2.2521x over previous
import functools

import jax
import jax.numpy as jnp
from jax import lax
from jax.experimental import pallas as pl
from jax.experimental.pallas import tpu as pltpu
from jax.experimental.pallas import tpu_sc as plsc

DIM = 128
N_EDGES = 320000
NUM_CORES = 2
NUM_SUBCORES = 16
LANES = 16
NUM_WORKERS = NUM_CORES * NUM_SUBCORES
ROWS_PER_WORKER = N_EDGES // NUM_WORKERS
CHUNK = 80
NUM_CHUNKS = ROWS_PER_WORKER // CHUNK
NBUF = 5
GROUPS = NUM_CHUNKS // NBUF


def _sc_gather(positions, pe):
    mesh = plsc.VectorSubcoreMesh(core_axis_name="c", subcore_axis_name="s")

    @functools.partial(
        pl.kernel,
        out_type=jax.ShapeDtypeStruct((N_EDGES, DIM), jnp.float32),
        mesh=mesh,
        scratch_types=[
            pltpu.VMEM((ROWS_PER_WORKER,), jnp.float32),
            pltpu.VMEM((ROWS_PER_WORKER,), jnp.int32),
            pltpu.VMEM_SHARED((1000, DIM), jnp.float32),
            pltpu.VMEM((NBUF, CHUNK, DIM), jnp.float32),
            pltpu.SemaphoreType.DMA((NBUF,)),
            pltpu.SemaphoreType.DMA((NBUF,)),
        ],
    )
    def run(pos_hbm, pe_hbm, out_hbm, pos_v, idx_v, table_sh, rows, gsem, wsem):
        wid = lax.axis_index("s") * NUM_CORES + lax.axis_index("c")
        base = wid * ROWS_PER_WORKER

        @pl.when(lax.axis_index("s") == 0)
        def _():
            pltpu.sync_copy(pe_hbm.at[pl.ds(0, 1000)], table_sh)

        pltpu.sync_copy(pos_hbm.at[pl.ds(base, ROWS_PER_WORKER)], pos_v)

        @pl.loop(0, ROWS_PER_WORKER, step=LANES)
        def _(i):
            p = pos_v[pl.ds(i, LANES)]
            v = jnp.minimum(jnp.maximum((p * 999.0).astype(jnp.int32), 0), 999)
            idx_v[pl.ds(i, LANES)] = v

        plsc.subcore_barrier()

        def fire_gathers(g):
            return [
                pltpu.async_copy(
                    table_sh.at[idx_v.at[pl.ds((g * NBUF + b) * CHUNK, CHUNK)]],
                    rows.at[b],
                    gsem.at[b],
                )
                for b in range(NBUF)
            ]

        def drain_and_write(g, gathers):
            for b in range(NBUF):
                gathers[b].wait()
                pltpu.async_copy(
                    rows.at[b],
                    out_hbm.at[pl.ds(base + (g * NBUF + b) * CHUNK, CHUNK)],
                    wsem.at[b],
                )

        def drain_writes():
            for b in range(NBUF):
                pltpu.make_async_copy(
                    rows.at[b], out_hbm.at[pl.ds(base, CHUNK)], wsem.at[b]
                ).wait()

        drain_and_write(0, fire_gathers(0))

        @pl.loop(1, GROUPS)
        def _(g):
            drain_writes()
            drain_and_write(g, fire_gathers(g))

        drain_writes()

    return run(positions, pe)


def kernel(positions, pe):
    return _sc_gather(positions, pe)
